# baseline (speedup 1.0000x reference)
import functools

import jax
import jax.numpy as jnp
from jax import lax
from jax.experimental import pallas as pl
from jax.experimental.pallas import tpu as pltpu
from jax.experimental.pallas import tpu_sc as plsc

N = 10000
D = 128
E = 320000
NC = 2
NS = 16
NW = NC * NS
BATCH = 128
EW = E // NW
K = 80
EWP = K * BATCH
NP = 10240
RPT = NP // NS

_MESH = plsc.VectorSubcoreMesh(
    core_axis_name="c", subcore_axis_name="s", num_cores=NC, num_subcores=NS
)


@functools.partial(
    pl.kernel,
    out_type=jax.ShapeDtypeStruct((NC, NP, D), jnp.float32),
    mesh=_MESH,
    scratch_types=[
        pltpu.VMEM((K, BATCH), jnp.int32),
        pltpu.VMEM((K, BATCH), jnp.int32),
        pltpu.VMEM((BATCH, D), jnp.float32),
        pltpu.VMEM_SHARED((NP, D), jnp.float32),
        pltpu.SemaphoreType.DMA,
    ],
)
def _sc_aggregate(x_hbm, col_hbm, row_hbm, zero_hbm, out_hbm,
                  colv, rowv, rbuf, acc, sem):
    cid = lax.axis_index("c")
    sid = lax.axis_index("s")
    wid = sid * NC + cid

    pltpu.sync_copy(col_hbm.at[wid], colv)
    pltpu.sync_copy(row_hbm.at[wid], rowv)
    pltpu.sync_copy(zero_hbm.at[pl.ds(sid * RPT, RPT)],
                    acc.at[pl.ds(sid * RPT, RPT)])
    plsc.subcore_barrier()

    def step(j, carry):
        pltpu.async_copy(x_hbm.at[colv.at[j]], rbuf, sem).wait()
        pltpu.sync_copy(rbuf, acc.at[rowv.at[j]], add=True)
        return carry

    lax.fori_loop(0, K, step, 0)
    plsc.subcore_barrier()

    pltpu.sync_copy(acc.at[pl.ds(sid * RPT, RPT)],
                    out_hbm.at[cid, pl.ds(sid * RPT, RPT)])


def _mm_body(p_ref, w_ref, o_ref):
    s = p_ref[0] + p_ref[1]
    o_ref[...] = jnp.maximum(
        jnp.dot(s, w_ref[...], preferred_element_type=jnp.float32), 0.0)


_BM = 1000


def _tc_combine_matmul(p, w):
    return pl.pallas_call(
        _mm_body,
        grid=(N // _BM,),
        in_specs=[
            pl.BlockSpec((NC, _BM, D), lambda i: (0, i, 0)),
            pl.BlockSpec((D, D), lambda i: (0, 0)),
        ],
        out_specs=pl.BlockSpec((_BM, D), lambda i: (i, 0)),
        out_shape=jax.ShapeDtypeStruct((N, D), jnp.float32),
    )(p, w)


def _pad_edges(edge_index):
    npad = EWP - EW
    pad_col = jnp.zeros((NW, npad), jnp.int32)
    pad_row = jnp.broadcast_to(
        N + (jnp.arange(npad, dtype=jnp.int32) % (NP - N)), (NW, npad))
    col = jnp.concatenate([edge_index[1].reshape(NW, EW), pad_col], axis=1)
    row = jnp.concatenate([edge_index[0].reshape(NW, EW), pad_row], axis=1)
    return col.reshape(NW, K, BATCH), row.reshape(NW, K, BATCH)


def kernel(x, edge_index0, edge_index1, W0, W1):
    col0, row0 = _pad_edges(edge_index0)
    col1, row1 = _pad_edges(edge_index1)
    zero = jnp.zeros((NP, D), jnp.float32)

    p0 = _sc_aggregate(x, col0, row0, zero)
    h1 = _tc_combine_matmul(p0, W0)
    p1 = _sc_aggregate(h1, col1, row1, zero)
    return _tc_combine_matmul(p1, W1)


# speedup vs baseline: 1.5992x; 1.5992x over previous
import functools

import jax
import jax.numpy as jnp
from jax import lax
from jax.experimental import pallas as pl
from jax.experimental.pallas import tpu as pltpu
from jax.experimental.pallas import tpu_sc as plsc

N = 10000
D = 128
E = 320000
NC = 2
NS = 16
NW = NC * NS
BATCH = 112
EW = E // NW
K = 90
EWP = K * BATCH
NP = 10240
RPT = NP // NS

_MESH = plsc.VectorSubcoreMesh(
    core_axis_name="c", subcore_axis_name="s", num_cores=NC, num_subcores=NS
)


@functools.partial(
    pl.kernel,
    out_type=jax.ShapeDtypeStruct((NC, NP, D), jnp.float32),
    mesh=_MESH,
    scratch_types=[
        pltpu.VMEM((K, BATCH), jnp.int32),
        pltpu.VMEM((K, BATCH), jnp.int32),
        pltpu.VMEM((BATCH, D), jnp.float32),
        pltpu.VMEM_SHARED((NP, D), jnp.float32),
        pltpu.SemaphoreType.DMA,
    ],
)
def _sc_aggregate(x_hbm, col_hbm, row_hbm, zero_hbm, out_hbm,
                  colv, rowv, rbuf, acc, sem):
    cid = lax.axis_index("c")
    sid = lax.axis_index("s")
    wid = sid * NC + cid

    pltpu.sync_copy(col_hbm.at[wid], colv)
    pltpu.sync_copy(row_hbm.at[wid], rowv)
    pltpu.sync_copy(zero_hbm.at[pl.ds(sid * RPT, RPT)],
                    acc.at[pl.ds(sid * RPT, RPT)])
    plsc.subcore_barrier()

    def step(j, carry):
        pltpu.async_copy(x_hbm.at[colv.at[j]], rbuf, sem).wait()
        pltpu.sync_copy(rbuf, acc.at[rowv.at[j]], add=True)
        return carry

    lax.fori_loop(0, K, step, 0)
    plsc.subcore_barrier()

    pltpu.sync_copy(acc.at[pl.ds(sid * RPT, RPT)],
                    out_hbm.at[cid, pl.ds(sid * RPT, RPT)])


def _mm_body(p_ref, w_ref, o_ref):
    s = p_ref[0] + p_ref[1]
    o_ref[...] = jnp.maximum(
        jnp.dot(s, w_ref[...], preferred_element_type=jnp.float32), 0.0)


_BM = 1000


def _tc_combine_matmul(p, w):
    return pl.pallas_call(
        _mm_body,
        grid=(N // _BM,),
        in_specs=[
            pl.BlockSpec((NC, _BM, D), lambda i: (0, i, 0)),
            pl.BlockSpec((D, D), lambda i: (0, 0)),
        ],
        out_specs=pl.BlockSpec((_BM, D), lambda i: (i, 0)),
        out_shape=jax.ShapeDtypeStruct((N, D), jnp.float32),
    )(p, w)


def _pad_edges(edge_index):
    npad = EWP - EW
    pad_col = jnp.zeros((NW, npad), jnp.int32)
    pad_row = jnp.broadcast_to(
        N + (jnp.arange(npad, dtype=jnp.int32) % (NP - N)), (NW, npad))
    col = jnp.concatenate([edge_index[1].reshape(NW, EW), pad_col], axis=1)
    row = jnp.concatenate([edge_index[0].reshape(NW, EW), pad_row], axis=1)
    return col.reshape(NW, K, BATCH), row.reshape(NW, K, BATCH)


def kernel(x, edge_index0, edge_index1, W0, W1):
    col0, row0 = _pad_edges(edge_index0)
    col1, row1 = _pad_edges(edge_index1)
    zero = jnp.zeros((NP, D), jnp.float32)

    p0 = _sc_aggregate(x, col0, row0, zero)
    h1 = _tc_combine_matmul(p0, W0)
    p1 = _sc_aggregate(h1, col1, row1, zero)
    return _tc_combine_matmul(p1, W1)


# speedup vs baseline: 2.2518x; 1.4081x over previous
import functools

import jax
import jax.numpy as jnp
from jax import lax
from jax.experimental import pallas as pl
from jax.experimental.pallas import tpu as pltpu
from jax.experimental.pallas import tpu_sc as plsc

N = 10000
D = 128
E = 320000
NC = 2
NS = 16
NW = NC * NS
BATCH = 100
EW = E // NW
K = 100
EWP = K * BATCH
NP = 10240
RPT = NP // NS

_MESH = plsc.VectorSubcoreMesh(
    core_axis_name="c", subcore_axis_name="s", num_cores=NC, num_subcores=NS
)


@functools.partial(
    pl.kernel,
    out_type=jax.ShapeDtypeStruct((NC, NP, D), jnp.float32),
    mesh=_MESH,
    scratch_types=[
        pltpu.VMEM((K, BATCH), jnp.int32),
        pltpu.VMEM((K, BATCH), jnp.int32),
        pltpu.VMEM((BATCH, D), jnp.float32),
        pltpu.VMEM_SHARED((NP, D), jnp.float32),
        pltpu.SemaphoreType.DMA,
    ],
)
def _sc_aggregate(x_hbm, col_hbm, row_hbm, zero_hbm, out_hbm,
                  colv, rowv, rbuf, acc, sem):
    cid = lax.axis_index("c")
    sid = lax.axis_index("s")
    wid = sid * NC + cid

    pltpu.sync_copy(col_hbm.at[wid], colv)
    pltpu.sync_copy(row_hbm.at[wid], rowv)
    pltpu.sync_copy(zero_hbm.at[pl.ds(sid * RPT, RPT)],
                    acc.at[pl.ds(sid * RPT, RPT)])
    plsc.subcore_barrier()

    def step(j, carry):
        pltpu.async_copy(x_hbm.at[colv.at[j]], rbuf, sem).wait()
        pltpu.sync_copy(rbuf, acc.at[rowv.at[j]], add=True)
        return carry

    lax.fori_loop(0, K, step, 0)
    plsc.subcore_barrier()

    pltpu.sync_copy(acc.at[pl.ds(sid * RPT, RPT)],
                    out_hbm.at[cid, pl.ds(sid * RPT, RPT)])


def _mm_body(p_ref, w_ref, o_ref):
    s = p_ref[0] + p_ref[1]
    o_ref[...] = jnp.maximum(
        jnp.dot(s, w_ref[...], preferred_element_type=jnp.float32), 0.0)


_BM = 1000


def _tc_combine_matmul(p, w):
    return pl.pallas_call(
        _mm_body,
        grid=(N // _BM,),
        in_specs=[
            pl.BlockSpec((NC, _BM, D), lambda i: (0, i, 0)),
            pl.BlockSpec((D, D), lambda i: (0, 0)),
        ],
        out_specs=pl.BlockSpec((_BM, D), lambda i: (i, 0)),
        out_shape=jax.ShapeDtypeStruct((N, D), jnp.float32),
    )(p, w)


def _pad_edges(edge_index):
    npad = EWP - EW
    pad_col = jnp.zeros((NW, npad), jnp.int32)
    pad_row = jnp.broadcast_to(
        N + (jnp.arange(npad, dtype=jnp.int32) % (NP - N)), (NW, npad))
    col = jnp.concatenate([edge_index[1].reshape(NW, EW), pad_col], axis=1)
    row = jnp.concatenate([edge_index[0].reshape(NW, EW), pad_row], axis=1)
    return col.reshape(NW, K, BATCH), row.reshape(NW, K, BATCH)


def kernel(x, edge_index0, edge_index1, W0, W1):
    col0, row0 = _pad_edges(edge_index0)
    col1, row1 = _pad_edges(edge_index1)
    zero = jnp.zeros((NP, D), jnp.float32)

    p0 = _sc_aggregate(x, col0, row0, zero)
    h1 = _tc_combine_matmul(p0, W0)
    p1 = _sc_aggregate(h1, col1, row1, zero)
    return _tc_combine_matmul(p1, W1)


# speedup vs baseline: 2.3934x; 1.0629x over previous
import functools

import jax
import jax.numpy as jnp
from jax import lax
from jax.experimental import pallas as pl
from jax.experimental.pallas import tpu as pltpu
from jax.experimental.pallas import tpu_sc as plsc

N = 10000
D = 128
E = 320000
NC = 2
NS = 16
NW = NC * NS
BATCH = 125
EW = E // NW
K = 80
EWP = K * BATCH
NP = 10240
RPT = NP // NS

_MESH = plsc.VectorSubcoreMesh(
    core_axis_name="c", subcore_axis_name="s", num_cores=NC, num_subcores=NS
)


@functools.partial(
    pl.kernel,
    out_type=jax.ShapeDtypeStruct((NC, NP, D), jnp.float32),
    mesh=_MESH,
    scratch_types=[
        pltpu.VMEM((K, BATCH), jnp.int32),
        pltpu.VMEM((K, BATCH), jnp.int32),
        pltpu.VMEM((BATCH, D), jnp.float32),
        pltpu.VMEM_SHARED((NP, D), jnp.float32),
        pltpu.SemaphoreType.DMA,
    ],
)
def _sc_aggregate(x_hbm, col_hbm, row_hbm, zero_hbm, out_hbm,
                  colv, rowv, rbuf, acc, sem):
    cid = lax.axis_index("c")
    sid = lax.axis_index("s")
    wid = sid * NC + cid

    pltpu.sync_copy(col_hbm.at[wid], colv)
    pltpu.sync_copy(row_hbm.at[wid], rowv)
    pltpu.sync_copy(zero_hbm.at[pl.ds(sid * RPT, RPT)],
                    acc.at[pl.ds(sid * RPT, RPT)])
    plsc.subcore_barrier()

    def step(j, carry):
        pltpu.async_copy(x_hbm.at[colv.at[j]], rbuf, sem).wait()
        pltpu.sync_copy(rbuf, acc.at[rowv.at[j]], add=True)
        return carry

    lax.fori_loop(0, K, step, 0)
    plsc.subcore_barrier()

    pltpu.sync_copy(acc.at[pl.ds(sid * RPT, RPT)],
                    out_hbm.at[cid, pl.ds(sid * RPT, RPT)])


def _mm_body(p_ref, w_ref, o_ref):
    s = p_ref[0] + p_ref[1]
    o_ref[...] = jnp.maximum(
        jnp.dot(s, w_ref[...], preferred_element_type=jnp.float32), 0.0)


_BM = 1000


def _tc_combine_matmul(p, w):
    return pl.pallas_call(
        _mm_body,
        grid=(N // _BM,),
        in_specs=[
            pl.BlockSpec((NC, _BM, D), lambda i: (0, i, 0)),
            pl.BlockSpec((D, D), lambda i: (0, 0)),
        ],
        out_specs=pl.BlockSpec((_BM, D), lambda i: (i, 0)),
        out_shape=jax.ShapeDtypeStruct((N, D), jnp.float32),
    )(p, w)


def _pad_edges(edge_index):
    npad = EWP - EW
    pad_col = jnp.zeros((NW, npad), jnp.int32)
    pad_row = jnp.broadcast_to(
        N + (jnp.arange(npad, dtype=jnp.int32) % (NP - N)), (NW, npad))
    col = jnp.concatenate([edge_index[1].reshape(NW, EW), pad_col], axis=1)
    row = jnp.concatenate([edge_index[0].reshape(NW, EW), pad_row], axis=1)
    return col.reshape(NW, K, BATCH), row.reshape(NW, K, BATCH)


def kernel(x, edge_index0, edge_index1, W0, W1):
    col0, row0 = _pad_edges(edge_index0)
    col1, row1 = _pad_edges(edge_index1)
    zero = jnp.zeros((NP, D), jnp.float32)

    p0 = _sc_aggregate(x, col0, row0, zero)
    h1 = _tc_combine_matmul(p0, W0)
    p1 = _sc_aggregate(h1, col1, row1, zero)
    return _tc_combine_matmul(p1, W1)
